# baseline (speedup 1.0000x reference)
import functools

import jax
import jax.numpy as jnp
from jax import lax
from jax.experimental import pallas as pl
from jax.experimental.pallas import tpu as pltpu
from jax.experimental.pallas import tpu_sc as plsc

N_NODES = 10000
N_EDGES = 160000
D_FEAT = 256

NC = 2
NT = 16
LANES = 16

NPAD = 10240
NODES_PT = NPAD // NT
EPAD = 163840
EDGES_PT = EPAD // NT
CHUNK = 128
NCHUNKS = EDGES_PT // CHUNK
NODE_CH = NODES_PT // CHUNK
DEG_ROWS = 16
NQ = 4
DQ = D_FEAT // NQ
VPR = DQ // LANES


def _rsqrt_nr(d):
    ds = jnp.maximum(d, 1.0)
    i = plsc.bitcast(ds, jnp.int32)
    i = 0x5F3759DF - lax.shift_right_arithmetic(i, 1)
    y = plsc.bitcast(i, jnp.float32)
    for _ in range(3):
        y = y * (1.5 - 0.5 * ds * y * y)
    return jnp.where(d > 0.0, y, 0.0)


def _sc_body(xq, rows, cols, out, y_hbm, deg_v, dinv_v, cbig,
             ridx_all, cidx_all, g0, g1, acc_sh, sem0, sem1):
    c = lax.axis_index("c")
    s = lax.axis_index("s")
    nbase = s * NODES_PT

    zv = jnp.zeros((LANES,), jnp.float32)
    ones = jnp.ones((LANES,), jnp.float32)
    gbufs = (g0, g1)
    sems = (sem0, sem1)
    dummy = xq.at[0, pl.ds(0, CHUNK)]

    def zero_g0():
        def zrow(r, _):
            for k in range(VPR):
                g0[r, pl.ds(k * LANES, LANES)] = zv
            return 0
        lax.fori_loop(0, CHUNK, zrow, 0)

    pltpu.sync_copy(rows.at[pl.ds(s * NCHUNKS, NCHUNKS)], ridx_all)
    pltpu.sync_copy(cols.at[pl.ds(s * NCHUNKS, NCHUNKS)], cidx_all)
    ybase = c * NPAD

    def roff(i, _):
        for k in range(CHUNK // LANES):
            sl = pl.ds(k * LANES, LANES)
            ridx_all[i, sl] = ridx_all[i, sl] + ybase
        return 0
    lax.fori_loop(0, NCHUNKS, roff, 0)

    def zdeg(i, _):
        deg_v[pl.ds(i * LANES, LANES)] = zv
        return 0
    lax.fori_loop(0, NODES_PT // LANES, zdeg, 0)

    def dchunk(j, _):
        pltpu.sync_copy(cols.at[pl.ds(j * DEG_ROWS, DEG_ROWS)], cbig)

        def drow(r, _):
            for k in range(CHUNK // LANES):
                cv = cbig[r, pl.ds(k * LANES, LANES)]
                loc = cv - nbase
                m = (loc >= 0) & (loc < NODES_PT)
                idx = jnp.where(m, loc, 0)
                plsc.addupdate_scatter(deg_v, [idx], ones, mask=m)
            return 0
        lax.fori_loop(0, DEG_ROWS, drow, 0)
        return 0
    lax.fori_loop(0, EPAD // (DEG_ROWS * CHUNK), dchunk, 0)

    def inv1(i, _):
        sl = pl.ds(i * LANES, LANES)
        dinv_v[sl] = _rsqrt_nr(deg_v[sl])
        return 0
    lax.fori_loop(0, NODES_PT // LANES, inv1, 0)

    for p in range(2):
        q = c * 2 + p

        for zi in range(NODE_CH):
            rb = zi * CHUNK
            pltpu.sync_copy(xq.at[q, pl.ds(nbase + rb, CHUNK)], g0)

            def scale_grp(i, _):
                dvec = dinv_v[pl.ds(rb + i * LANES, LANES)]
                for j in range(LANES):
                    sc = dvec[j]
                    n = i * LANES + j
                    for k in range(VPR):
                        sl = pl.ds(k * LANES, LANES)
                        g0[n, sl] = g0[n, sl] * sc
                return 0
            lax.fori_loop(0, CHUNK // LANES, scale_grp, 0)
            pltpu.sync_copy(g0, y_hbm.at[pl.ds(ybase + nbase + rb, CHUNK)])

        zero_g0()
        for zi in range(NODE_CH):
            pltpu.sync_copy(g0, acc_sh.at[pl.ds(nbase + zi * CHUNK, CHUNK)])
        plsc.subcore_barrier()

        for l in range(2):
            pltpu.async_copy(y_hbm.at[ridx_all.at[0]], g0, sem0)

            def epair(j2, _):
                jj = j2 * 2
                for b in range(2):
                    nxt = jj + b + 1

                    @pl.when(nxt < NCHUNKS)
                    def _():
                        pltpu.async_copy(
                            y_hbm.at[ridx_all.at[nxt]],
                            gbufs[1 - b], sems[1 - b])
                    pltpu.make_async_copy(dummy, gbufs[b], sems[b]).wait()
                    pltpu.sync_copy(
                        gbufs[b], acc_sh.at[cidx_all.at[jj + b]], add=True)
                return 0
            lax.fori_loop(0, NCHUNKS // 2, epair, 0)
            plsc.subcore_barrier()

            if l == 0:
                for zi in range(NODE_CH):
                    rb = zi * CHUNK
                    pltpu.sync_copy(acc_sh.at[pl.ds(nbase + rb, CHUNK)], g0)

                    def mid_grp(i, _):
                        dvec = dinv_v[pl.ds(rb + i * LANES, LANES)]
                        for j in range(LANES):
                            sc = dvec[j]
                            n = i * LANES + j
                            for k in range(VPR):
                                sl = pl.ds(k * LANES, LANES)
                                v = g0[n, sl]
                                g0[n, sl] = jnp.maximum(v * sc, 0.0) * sc
                        return 0
                    lax.fori_loop(0, CHUNK // LANES, mid_grp, 0)
                    pltpu.sync_copy(
                        g0, y_hbm.at[pl.ds(ybase + nbase + rb, CHUNK)])

                zero_g0()
                for zi in range(NODE_CH):
                    pltpu.sync_copy(
                        g0, acc_sh.at[pl.ds(nbase + zi * CHUNK, CHUNK)])
                plsc.subcore_barrier()
            else:
                for zi in range(NODE_CH):
                    rb = zi * CHUNK
                    pltpu.sync_copy(acc_sh.at[pl.ds(nbase + rb, CHUNK)], g0)

                    def out_grp(i, _):
                        dvec = dinv_v[pl.ds(rb + i * LANES, LANES)]
                        for j in range(LANES):
                            sc = dvec[j]
                            n = i * LANES + j
                            for k in range(VPR):
                                sl = pl.ds(k * LANES, LANES)
                                g0[n, sl] = jnp.maximum(g0[n, sl] * sc, 0.0)
                        return 0
                    lax.fori_loop(0, CHUNK // LANES, out_grp, 0)
                    pltpu.sync_copy(
                        g0, out.at[q, pl.ds(nbase + rb, CHUNK)])
                if p == 0:
                    plsc.subcore_barrier()


_sc_call = functools.partial(
    pl.kernel,
    out_type=(jax.ShapeDtypeStruct((NQ, NPAD, DQ), jnp.float32),
              jax.ShapeDtypeStruct((NC * NPAD, DQ), jnp.float32)),
    mesh=plsc.VectorSubcoreMesh(
        core_axis_name="c", subcore_axis_name="s",
        num_cores=NC, num_subcores=NT),
    scratch_types=[
        pltpu.VMEM((NODES_PT,), jnp.float32),
        pltpu.VMEM((NODES_PT,), jnp.float32),
        pltpu.VMEM((DEG_ROWS, CHUNK), jnp.int32),
        pltpu.VMEM((NCHUNKS, CHUNK), jnp.int32),
        pltpu.VMEM((NCHUNKS, CHUNK), jnp.int32),
        pltpu.VMEM((CHUNK, DQ), jnp.float32),
        pltpu.VMEM((CHUNK, DQ), jnp.float32),
        pltpu.VMEM_SHARED((NPAD, DQ), jnp.float32),
        pltpu.SemaphoreType.DMA,
        pltpu.SemaphoreType.DMA,
    ],
    compiler_params=pltpu.CompilerParams(
        needs_layout_passes=False, use_tc_tiling_on_sc=False),
)(_sc_body)


@jax.jit
def kernel(x, edge_index):
    row = edge_index[0].astype(jnp.int32)
    col = edge_index[1].astype(jnp.int32)
    xp = jnp.zeros((NPAD, D_FEAT), jnp.float32).at[:N_NODES].set(x)
    xq = xp.reshape(NPAD, NQ, DQ).transpose(1, 0, 2)
    rp = jnp.full((EPAD,), N_NODES, jnp.int32).at[:N_EDGES].set(row)
    cp = jnp.full((EPAD,), NPAD - 1, jnp.int32).at[:N_EDGES].set(col)
    outq, _ = _sc_call(
        xq, rp.reshape(EPAD // CHUNK, CHUNK), cp.reshape(EPAD // CHUNK, CHUNK))
    return outq.transpose(1, 0, 2).reshape(NPAD, D_FEAT)[:N_NODES]


# speedup vs baseline: 1.8791x; 1.8791x over previous
import functools

import jax
import jax.numpy as jnp
from jax import lax
from jax.experimental import pallas as pl
from jax.experimental.pallas import tpu as pltpu
from jax.experimental.pallas import tpu_sc as plsc

N_NODES = 10000
N_EDGES = 160000
D_FEAT = 256

NC = 2
NT = 16
LANES = 16

NPAD = 10240
NODES_PT = NPAD // NT
EPAD = 163840
EDGES_PT = EPAD // NT
CHUNK = 128
NCHUNKS = EDGES_PT // CHUNK
NODE_CH = NODES_PT // CHUNK
NQ = 4
DQ = D_FEAT // NQ
VPR = DQ // LANES
NBUF = 3
MAIN = (NCHUNKS // NBUF) * NBUF - NBUF


def _rsqrt_nr(d):
    ds = jnp.maximum(d, 1.0)
    i = plsc.bitcast(ds, jnp.int32)
    i = 0x5F3759DF - lax.shift_right_arithmetic(i, 1)
    y = plsc.bitcast(i, jnp.float32)
    for _ in range(3):
        y = y * (1.5 - 0.5 * ds * y * y)
    return jnp.where(d > 0.0, y, 0.0)


def _sc_body(xq, rows, cols, out, deg_v, dinv_v, ridx_all, cidx_all,
             g0, g1, g2, y_sh, acc_sh, gs0, gs1, gs2, ss0, ss1, ss2):
    c = lax.axis_index("c")
    s = lax.axis_index("s")
    nbase = s * NODES_PT

    zv = jnp.zeros((LANES,), jnp.float32)
    ones = jnp.ones((LANES,), jnp.float32)
    gbufs = (g0, g1, g2)
    gsems = (gs0, gs1, gs2)
    ssems = (ss0, ss1, ss2)
    dummy = xq.at[0, pl.ds(0, CHUNK)]

    def zero_g0():
        def zrow(r, _):
            for k in range(VPR):
                g0[r, pl.ds(k * LANES, LANES)] = zv
            return 0
        lax.fori_loop(0, CHUNK, zrow, 0)

    def zdeg(i, _):
        deg_v[pl.ds(i * LANES, LANES)] = zv
        return 0
    lax.fori_loop(0, NODES_PT // LANES, zdeg, 0)

    def dchunk(j, _):
        pltpu.sync_copy(cols.at[pl.ds(j * NCHUNKS, NCHUNKS)], ridx_all)

        def drow(r, _):
            for k in range(CHUNK // LANES):
                cv = ridx_all[r, pl.ds(k * LANES, LANES)]
                loc = cv - nbase
                m = (loc >= 0) & (loc < NODES_PT)
                idx = jnp.where(m, loc, 0)
                plsc.addupdate_scatter(deg_v, [idx], ones, mask=m)
            return 0
        lax.fori_loop(0, NCHUNKS, drow, 0)
        return 0
    lax.fori_loop(0, EPAD // (NCHUNKS * CHUNK), dchunk, 0)

    def inv1(i, _):
        sl = pl.ds(i * LANES, LANES)
        dinv_v[sl] = _rsqrt_nr(deg_v[sl])
        return 0
    lax.fori_loop(0, NODES_PT // LANES, inv1, 0)

    pltpu.sync_copy(rows.at[pl.ds(s * NCHUNKS, NCHUNKS)], ridx_all)
    pltpu.sync_copy(cols.at[pl.ds(s * NCHUNKS, NCHUNKS)], cidx_all)

    def edge_chunk(jj, b, first):
        pltpu.make_async_copy(dummy, gbufs[b], gsems[b]).wait()
        pltpu.async_copy(
            gbufs[b], acc_sh.at[cidx_all.at[jj]], ssems[b], add=True)
        if first:
            pltpu.async_copy(
                y_sh.at[ridx_all.at[jj + NBUF - 1]],
                gbufs[(b + 2) % NBUF], gsems[(b + 2) % NBUF])
        else:
            pltpu.make_async_copy(
                dummy, gbufs[(b + 2) % NBUF], ssems[(b + 2) % NBUF]).wait()

            @pl.when(jj + NBUF - 1 < NCHUNKS)
            def _():
                pltpu.async_copy(
                    y_sh.at[ridx_all.at[jj + NBUF - 1]],
                    gbufs[(b + 2) % NBUF], gsems[(b + 2) % NBUF])

    for p in range(2):
        q = c * 2 + p

        for zi in range(NODE_CH):
            rb = zi * CHUNK
            pltpu.sync_copy(xq.at[q, pl.ds(nbase + rb, CHUNK)], g0)

            def scale_grp(i, _):
                dvec = dinv_v[pl.ds(rb + i * LANES, LANES)]
                for j in range(LANES):
                    sc = dvec[j]
                    n = i * LANES + j
                    for k in range(VPR):
                        sl = pl.ds(k * LANES, LANES)
                        g0[n, sl] = g0[n, sl] * sc
                return 0
            lax.fori_loop(0, CHUNK // LANES, scale_grp, 0)
            pltpu.sync_copy(g0, y_sh.at[pl.ds(nbase + rb, CHUNK)])

        zero_g0()
        for zi in range(NODE_CH):
            pltpu.sync_copy(g0, acc_sh.at[pl.ds(nbase + zi * CHUNK, CHUNK)])
        plsc.subcore_barrier()

        for l in range(2):
            pltpu.async_copy(y_sh.at[ridx_all.at[0]], g0, gs0)
            pltpu.async_copy(y_sh.at[ridx_all.at[1]], g1, gs1)
            for b in range(NBUF):
                edge_chunk(b, b, first=(b == 0))

            def etri(j3, _):
                for b in range(NBUF):
                    edge_chunk(NBUF + j3 * NBUF + b, b, first=False)
                return 0
            lax.fori_loop(0, MAIN // NBUF, etri, 0)
            for t in range(NBUF + MAIN, NCHUNKS):
                edge_chunk(t, t % NBUF, first=False)
            lb = (NCHUNKS - 1) % NBUF
            pltpu.make_async_copy(dummy, gbufs[lb], ssems[lb]).wait()
            plsc.subcore_barrier()

            if l == 0:
                for zi in range(NODE_CH):
                    rb = zi * CHUNK
                    pltpu.sync_copy(acc_sh.at[pl.ds(nbase + rb, CHUNK)], g0)

                    def mid_grp(i, _):
                        dvec = dinv_v[pl.ds(rb + i * LANES, LANES)]
                        for j in range(LANES):
                            sc = dvec[j]
                            n = i * LANES + j
                            for k in range(VPR):
                                sl = pl.ds(k * LANES, LANES)
                                v = g0[n, sl]
                                g0[n, sl] = jnp.maximum(v * sc, 0.0) * sc
                        return 0
                    lax.fori_loop(0, CHUNK // LANES, mid_grp, 0)
                    pltpu.sync_copy(g0, y_sh.at[pl.ds(nbase + rb, CHUNK)])

                zero_g0()
                for zi in range(NODE_CH):
                    pltpu.sync_copy(
                        g0, acc_sh.at[pl.ds(nbase + zi * CHUNK, CHUNK)])
                plsc.subcore_barrier()
            else:
                for zi in range(NODE_CH):
                    rb = zi * CHUNK
                    pltpu.sync_copy(acc_sh.at[pl.ds(nbase + rb, CHUNK)], g0)

                    def out_grp(i, _):
                        dvec = dinv_v[pl.ds(rb + i * LANES, LANES)]
                        for j in range(LANES):
                            sc = dvec[j]
                            n = i * LANES + j
                            for k in range(VPR):
                                sl = pl.ds(k * LANES, LANES)
                                g0[n, sl] = jnp.maximum(g0[n, sl] * sc, 0.0)
                        return 0
                    lax.fori_loop(0, CHUNK // LANES, out_grp, 0)
                    pltpu.sync_copy(
                        g0, out.at[q, pl.ds(nbase + rb, CHUNK)])
                if p == 0:
                    plsc.subcore_barrier()


_sc_call = functools.partial(
    pl.kernel,
    out_type=jax.ShapeDtypeStruct((NQ, NPAD, DQ), jnp.float32),
    mesh=plsc.VectorSubcoreMesh(
        core_axis_name="c", subcore_axis_name="s",
        num_cores=NC, num_subcores=NT),
    scratch_types=[
        pltpu.VMEM((NODES_PT,), jnp.float32),
        pltpu.VMEM((NODES_PT,), jnp.float32),
        pltpu.VMEM((NCHUNKS, CHUNK), jnp.int32),
        pltpu.VMEM((NCHUNKS, CHUNK), jnp.int32),
        pltpu.VMEM((CHUNK, DQ), jnp.float32),
        pltpu.VMEM((CHUNK, DQ), jnp.float32),
        pltpu.VMEM((CHUNK, DQ), jnp.float32),
        pltpu.VMEM_SHARED((NPAD, DQ), jnp.float32),
        pltpu.VMEM_SHARED((NPAD, DQ), jnp.float32),
        pltpu.SemaphoreType.DMA,
        pltpu.SemaphoreType.DMA,
        pltpu.SemaphoreType.DMA,
        pltpu.SemaphoreType.DMA,
        pltpu.SemaphoreType.DMA,
        pltpu.SemaphoreType.DMA,
    ],
    compiler_params=pltpu.CompilerParams(
        needs_layout_passes=False, use_tc_tiling_on_sc=False),
)(_sc_body)


@jax.jit
def kernel(x, edge_index):
    row = edge_index[0].astype(jnp.int32)
    col = edge_index[1].astype(jnp.int32)
    xp = jnp.zeros((NPAD, D_FEAT), jnp.float32).at[:N_NODES].set(x)
    xq = xp.reshape(NPAD, NQ, DQ).transpose(1, 0, 2)
    rp = jnp.full((EPAD,), N_NODES, jnp.int32).at[:N_EDGES].set(row)
    cp = jnp.full((EPAD,), NPAD - 1, jnp.int32).at[:N_EDGES].set(col)
    outq = _sc_call(
        xq, rp.reshape(EPAD // CHUNK, CHUNK), cp.reshape(EPAD // CHUNK, CHUNK))
    return outq.transpose(1, 0, 2).reshape(NPAD, D_FEAT)[:N_NODES]


# speedup vs baseline: 2.1297x; 1.1334x over previous
import functools

import jax
import jax.numpy as jnp
from jax import lax
from jax.experimental import pallas as pl
from jax.experimental.pallas import tpu as pltpu
from jax.experimental.pallas import tpu_sc as plsc

N_NODES = 10000
N_EDGES = 160000
D_FEAT = 256

NC = 2
NT = 16
LANES = 16

NPAD = 10240
NODES_PT = NPAD // NT
EPAD = 163840
EDGES_PT = EPAD // NT
CHUNK = 128
NCHUNKS = EDGES_PT // CHUNK
NODE_CH = NODES_PT // CHUNK
NQ = 4
DQ = D_FEAT // NQ
VPR = DQ // LANES
NBUF = 3
MAIN = (NCHUNKS // NBUF) * NBUF - NBUF


def _rsqrt_nr(d):
    ds = jnp.maximum(d, 1.0)
    i = plsc.bitcast(ds, jnp.int32)
    i = 0x5F3759DF - lax.shift_right_arithmetic(i, 1)
    y = plsc.bitcast(i, jnp.float32)
    for _ in range(3):
        y = y * (1.5 - 0.5 * ds * y * y)
    return jnp.where(d > 0.0, y, 0.0)


def _sc_body(xq, rows, cols, out, deg_v, dinv_v, ridx_all, cidx_all,
             g0, g1, g2, y_sh, acc_sh, gs0, gs1, gs2, ss0, ss1, ss2):
    c = lax.axis_index("c")
    s = lax.axis_index("s")
    nbase = s * NODES_PT

    zv = jnp.zeros((LANES,), jnp.float32)
    ones = jnp.ones((LANES,), jnp.float32)
    gbufs = (g0, g1, g2)
    gsems = (gs0, gs1, gs2)
    ssems = (ss0, ss1, ss2)
    dummy = xq.at[pl.ds(0, CHUNK), pl.ds(0, DQ)]

    def zero_g0():
        def zrow(r, _):
            for k in range(VPR):
                g0[r, pl.ds(k * LANES, LANES)] = zv
            return 0
        lax.fori_loop(0, CHUNK, zrow, 0)

    def zdeg(i, _):
        deg_v[pl.ds(i * LANES, LANES)] = zv
        return 0
    lax.fori_loop(0, NODES_PT // LANES, zdeg, 0)

    def dchunk(j, _):
        pltpu.sync_copy(cols.at[pl.ds(j * NCHUNKS, NCHUNKS)], ridx_all)

        def drow(r, _):
            for k in range(CHUNK // LANES):
                cv = ridx_all[r, pl.ds(k * LANES, LANES)]
                loc = cv - nbase
                m = (loc >= 0) & (loc < NODES_PT)
                idx = jnp.where(m, loc, 0)
                plsc.addupdate_scatter(deg_v, [idx], ones, mask=m)
            return 0
        lax.fori_loop(0, NCHUNKS, drow, 0)
        return 0
    lax.fori_loop(0, EPAD // (NCHUNKS * CHUNK), dchunk, 0)

    def inv1(i, _):
        sl = pl.ds(i * LANES, LANES)
        dinv_v[sl] = _rsqrt_nr(deg_v[sl])
        return 0
    lax.fori_loop(0, NODES_PT // LANES, inv1, 0)

    pltpu.sync_copy(rows.at[pl.ds(s * NCHUNKS, NCHUNKS)], ridx_all)
    pltpu.sync_copy(cols.at[pl.ds(s * NCHUNKS, NCHUNKS)], cidx_all)

    def edge_chunk(jj, b, first):
        pltpu.make_async_copy(dummy, gbufs[b], gsems[b]).wait()
        pltpu.async_copy(
            gbufs[b], acc_sh.at[cidx_all.at[jj]], ssems[b], add=True)
        if first:
            pltpu.async_copy(
                y_sh.at[ridx_all.at[jj + NBUF - 1]],
                gbufs[(b + 2) % NBUF], gsems[(b + 2) % NBUF])
        else:
            pltpu.make_async_copy(
                dummy, gbufs[(b + 2) % NBUF], ssems[(b + 2) % NBUF]).wait()

            @pl.when(jj + NBUF - 1 < NCHUNKS)
            def _():
                pltpu.async_copy(
                    y_sh.at[ridx_all.at[jj + NBUF - 1]],
                    gbufs[(b + 2) % NBUF], gsems[(b + 2) % NBUF])

    for p in range(2):
        q = c * 2 + p

        for zi in range(NODE_CH):
            rb = zi * CHUNK
            pltpu.sync_copy(
                xq.at[pl.ds(nbase + rb, CHUNK), pl.ds(q * DQ, DQ)], g0)

            def scale_grp(i, _):
                dvec = dinv_v[pl.ds(rb + i * LANES, LANES)]
                for j in range(LANES):
                    sc = dvec[j]
                    n = i * LANES + j
                    for k in range(VPR):
                        sl = pl.ds(k * LANES, LANES)
                        g0[n, sl] = g0[n, sl] * sc
                return 0
            lax.fori_loop(0, CHUNK // LANES, scale_grp, 0)
            pltpu.sync_copy(g0, y_sh.at[pl.ds(nbase + rb, CHUNK)])

        zero_g0()
        for zi in range(NODE_CH):
            pltpu.sync_copy(g0, acc_sh.at[pl.ds(nbase + zi * CHUNK, CHUNK)])
        plsc.subcore_barrier()

        for l in range(2):
            pltpu.async_copy(y_sh.at[ridx_all.at[0]], g0, gs0)
            pltpu.async_copy(y_sh.at[ridx_all.at[1]], g1, gs1)
            for b in range(NBUF):
                edge_chunk(b, b, first=(b == 0))

            def etri(j3, _):
                for b in range(NBUF):
                    edge_chunk(NBUF + j3 * NBUF + b, b, first=False)
                return 0
            lax.fori_loop(0, MAIN // NBUF, etri, 0)
            for t in range(NBUF + MAIN, NCHUNKS):
                edge_chunk(t, t % NBUF, first=False)
            lb = (NCHUNKS - 1) % NBUF
            pltpu.make_async_copy(dummy, gbufs[lb], ssems[lb]).wait()
            plsc.subcore_barrier()

            if l == 0:
                for zi in range(NODE_CH):
                    rb = zi * CHUNK
                    pltpu.sync_copy(acc_sh.at[pl.ds(nbase + rb, CHUNK)], g0)

                    def mid_grp(i, _):
                        dvec = dinv_v[pl.ds(rb + i * LANES, LANES)]
                        for j in range(LANES):
                            sc = dvec[j]
                            n = i * LANES + j
                            for k in range(VPR):
                                sl = pl.ds(k * LANES, LANES)
                                v = g0[n, sl]
                                g0[n, sl] = jnp.maximum(v * sc, 0.0) * sc
                        return 0
                    lax.fori_loop(0, CHUNK // LANES, mid_grp, 0)
                    pltpu.sync_copy(g0, y_sh.at[pl.ds(nbase + rb, CHUNK)])

                zero_g0()
                for zi in range(NODE_CH):
                    pltpu.sync_copy(
                        g0, acc_sh.at[pl.ds(nbase + zi * CHUNK, CHUNK)])
                plsc.subcore_barrier()
            else:
                for zi in range(NODE_CH):
                    rb = zi * CHUNK
                    pltpu.sync_copy(acc_sh.at[pl.ds(nbase + rb, CHUNK)], g0)

                    def out_grp(i, _):
                        dvec = dinv_v[pl.ds(rb + i * LANES, LANES)]
                        for j in range(LANES):
                            sc = dvec[j]
                            n = i * LANES + j
                            for k in range(VPR):
                                sl = pl.ds(k * LANES, LANES)
                                g0[n, sl] = jnp.maximum(g0[n, sl] * sc, 0.0)
                        return 0
                    lax.fori_loop(0, CHUNK // LANES, out_grp, 0)
                    pltpu.sync_copy(
                        g0,
                        out.at[pl.ds(nbase + rb, CHUNK), pl.ds(q * DQ, DQ)])
                if p == 0:
                    plsc.subcore_barrier()


_sc_call = functools.partial(
    pl.kernel,
    out_type=jax.ShapeDtypeStruct((NPAD, D_FEAT), jnp.float32),
    mesh=plsc.VectorSubcoreMesh(
        core_axis_name="c", subcore_axis_name="s",
        num_cores=NC, num_subcores=NT),
    scratch_types=[
        pltpu.VMEM((NODES_PT,), jnp.float32),
        pltpu.VMEM((NODES_PT,), jnp.float32),
        pltpu.VMEM((NCHUNKS, CHUNK), jnp.int32),
        pltpu.VMEM((NCHUNKS, CHUNK), jnp.int32),
        pltpu.VMEM((CHUNK, DQ), jnp.float32),
        pltpu.VMEM((CHUNK, DQ), jnp.float32),
        pltpu.VMEM((CHUNK, DQ), jnp.float32),
        pltpu.VMEM_SHARED((NPAD, DQ), jnp.float32),
        pltpu.VMEM_SHARED((NPAD, DQ), jnp.float32),
        pltpu.SemaphoreType.DMA,
        pltpu.SemaphoreType.DMA,
        pltpu.SemaphoreType.DMA,
        pltpu.SemaphoreType.DMA,
        pltpu.SemaphoreType.DMA,
        pltpu.SemaphoreType.DMA,
    ],
    compiler_params=pltpu.CompilerParams(
        needs_layout_passes=False, use_tc_tiling_on_sc=False),
)(_sc_body)


@jax.jit
def kernel(x, edge_index):
    row = edge_index[0].astype(jnp.int32)
    col = edge_index[1].astype(jnp.int32)
    xp = jnp.zeros((NPAD, D_FEAT), jnp.float32).at[:N_NODES].set(x)
    rp = jnp.full((EPAD,), N_NODES, jnp.int32).at[:N_EDGES].set(row)
    cp = jnp.full((EPAD,), NPAD - 1, jnp.int32).at[:N_EDGES].set(col)
    outp = _sc_call(
        xp, rp.reshape(EPAD // CHUNK, CHUNK), cp.reshape(EPAD // CHUNK, CHUNK))
    return outp[:N_NODES]
